# baseline (speedup 1.0000x reference)
import functools

import jax
import jax.numpy as jnp
from jax import lax
from jax.experimental import pallas as pl
from jax.experimental.pallas import tpu as pltpu
from jax.experimental.pallas import tpu_sc as plsc


@functools.cache
def _make_gather_t(V, D, B):
  info = plsc.get_sparse_core_info()
  NC, NS, L = info.num_cores, info.num_subcores, info.num_lanes
  NW = NC * NS
  assert D % NW == 0 and B % L == 0
  rows_per_w = D // NW
  CH = min(4096, B)
  UN = 16
  n_ch = B // CH
  mesh = plsc.VectorSubcoreMesh(core_axis_name="c", subcore_axis_name="s")

  @functools.partial(
      pl.kernel,
      mesh=mesh,
      compiler_params=pltpu.CompilerParams(
          use_tc_tiling_on_sc=True, needs_layout_passes=False
      ),
      out_type=jax.ShapeDtypeStruct((D, B), jnp.float32),
      scratch_types=[
          pltpu.VMEM((V,), jnp.float32),
          pltpu.VMEM((B,), jnp.int32),
          pltpu.VMEM((2, CH), jnp.float32),
          pltpu.SemaphoreType.DMA,
          pltpu.SemaphoreType.DMA,
          pltpu.SemaphoreType.DMA,
      ],
  )
  def gather_kernel(
      ids_hbm, tt_hbm, out_hbm, row_v, ids_v, out_v, sem_ids, sem_row, sem_out
  ):
    wid = lax.axis_index("s") * NC + lax.axis_index("c")
    d0 = wid * rows_per_w

    def start_row(d):
      return [pltpu.async_copy(tt_hbm.at[d], row_v, sem_row)]

    NCH_IDS = 8
    idw = B // NCH_IDS
    rot = (wid % NCH_IDS) * idw
    cp_ids = []
    for k in range(NCH_IDS):
      off = lax.rem(rot + k * idw, B)
      cp_ids.append(
          pltpu.async_copy(
              ids_hbm.at[pl.ds(off, idw)], ids_v.at[pl.ds(off, idw)], sem_ids
          )
      )
    cp_row = start_row(d0)
    for cp in cp_ids:
      cp.wait()

    writes = []
    for i in range(rows_per_w):
      d = d0 + i
      for cp in cp_row:
        cp.wait()
      for cb in range(n_ch):
        k = i * n_ch + cb
        buf = k % 2
        if k >= 2:
          writes[k - 2].wait()

        @plsc.parallel_loop(0, CH, step=L, unroll=UN)
        def _(off, cb=cb, buf=buf):
          idx = ids_v[pl.ds(cb * CH + off, L)]
          out_v[buf, pl.ds(off, L)] = plsc.load_gather(row_v, [idx])
        if cb == n_ch - 1 and i + 1 < rows_per_w:
          cp_row = start_row(d + 1)
        writes.append(
            pltpu.async_copy(
                out_v.at[buf], out_hbm.at[d, pl.ds(cb * CH, CH)], sem_out
            )
        )
    for w in writes[-2:]:
      w.wait()

  return gather_kernel


def kernel(speaker_ids, embedding_table):
  B, = speaker_ids.shape
  V, D = embedding_table.shape
  ids = speaker_ids.astype(jnp.int32)
  out_t = _make_gather_t(V, D, B)(ids, embedding_table.T)
  return out_t.T


# speedup vs baseline: 1.0184x; 1.0184x over previous
import functools

import jax
import jax.numpy as jnp
from jax import lax
from jax.experimental import pallas as pl
from jax.experimental.pallas import tpu as pltpu
from jax.experimental.pallas import tpu_sc as plsc


@functools.cache
def _make_gather_t(V, D, B):
  info = plsc.get_sparse_core_info()
  NC, NS, L = info.num_cores, info.num_subcores, info.num_lanes
  NW = NC * NS
  assert D % NW == 0 and B % L == 0
  rows_per_w = D // NW
  CH = min(4096, B)
  UN = 8
  n_ch = B // CH
  mesh = plsc.VectorSubcoreMesh(core_axis_name="c", subcore_axis_name="s")

  @functools.partial(
      pl.kernel,
      mesh=mesh,
      compiler_params=pltpu.CompilerParams(
          use_tc_tiling_on_sc=True, needs_layout_passes=False
      ),
      out_type=jax.ShapeDtypeStruct((D, B), jnp.float32),
      scratch_types=[
          pltpu.VMEM((V,), jnp.float32),
          pltpu.VMEM((B,), jnp.int32),
          pltpu.VMEM((2, CH), jnp.float32),
          pltpu.SemaphoreType.DMA,
          pltpu.SemaphoreType.DMA,
          pltpu.SemaphoreType.DMA,
      ],
  )
  def gather_kernel(
      ids_hbm, tt_hbm, out_hbm, row_v, ids_v, out_v, sem_ids, sem_row, sem_out
  ):
    wid = lax.axis_index("s") * NC + lax.axis_index("c")
    d0 = wid * rows_per_w

    def start_row(d):
      return [pltpu.async_copy(tt_hbm.at[d], row_v, sem_row)]

    NCH_IDS = 8
    idw = B // NCH_IDS
    rot = (wid % NCH_IDS) * idw
    cp_ids = []
    for k in range(NCH_IDS):
      off = lax.rem(rot + k * idw, B)
      cp_ids.append(
          pltpu.async_copy(
              ids_hbm.at[pl.ds(off, idw)], ids_v.at[pl.ds(off, idw)], sem_ids
          )
      )
    cp_row = start_row(d0)
    for cp in cp_ids:
      cp.wait()

    writes = []
    for i in range(rows_per_w):
      d = d0 + i
      for cp in cp_row:
        cp.wait()
      for cb in range(n_ch):
        k = i * n_ch + cb
        buf = k % 2
        if k >= 2:
          writes[k - 2].wait()

        @plsc.parallel_loop(0, CH, step=L, unroll=UN)
        def _(off, cb=cb, buf=buf):
          idx = ids_v[pl.ds(cb * CH + off, L)]
          out_v[buf, pl.ds(off, L)] = plsc.load_gather(row_v, [idx])
        if cb == n_ch - 1 and i + 1 < rows_per_w:
          cp_row = start_row(d + 1)
        writes.append(
            pltpu.async_copy(
                out_v.at[buf], out_hbm.at[d, pl.ds(cb * CH, CH)], sem_out
            )
        )
    for w in writes[-2:]:
      w.wait()

  return gather_kernel


def kernel(speaker_ids, embedding_table):
  B, = speaker_ids.shape
  V, D = embedding_table.shape
  ids = speaker_ids.astype(jnp.int32)
  out_t = _make_gather_t(V, D, B)(ids, embedding_table.T)
  return out_t.T
